# baseline (speedup 1.0000x reference)
import functools

import jax
import jax.numpy as jnp
from jax import lax
from jax.experimental import pallas as pl
from jax.experimental.pallas import tpu as pltpu
from jax.experimental.pallas import tpu_sc as plsc

N = 10000
E = 320000
IN_DIM = 128
HID = 256
OUT_DIM = 128
L = 3

EP = 327680
BLK = 128
NBLK = EP // (16 * BLK)
BN = 2000
GRID = N // BN
PAD_ROWS = 8

_TILE_ROWS = 624
_ROW_CHUNKS = ((0, 128), (128, 128), (256, 128), (384, 128), (512, 112))
_EXTRA_R0 = 16 * _TILE_ROWS
_EXTRA_ALL = 24
_EXTRA_OUT = 16
_ICH = 32


def _sc_agg_build():
  mesh = plsc.VectorSubcoreMesh(core_axis_name="c", subcore_axis_name="s")
  out_type = [jax.ShapeDtypeStruct((2, N, 128), jnp.float32)]
  scratch = [
      pltpu.VMEM((_ICH, BLK), jnp.int32),
      pltpu.VMEM((_ICH, BLK), jnp.int32),
      pltpu.VMEM((BLK, 128), jnp.float32),
      pltpu.VMEM((BLK, 128), jnp.float32),
      pltpu.VMEM_SHARED((N + PAD_ROWS, 128), jnp.float32),
      pltpu.SemaphoreType.DMA,
      pltpu.SemaphoreType.DMA,
      pltpu.SemaphoreType.DMA,
      pltpu.SemaphoreType.DMA,
  ]

  @functools.partial(pl.kernel, mesh=mesh, out_type=out_type,
                     scratch_types=scratch)
  def agg(h2, src2, dst2, z128, *refs):
    agg_out, isrc, idst, g0, g1, acc, sem0, sem1, ssem0, ssem1 = refs
    c = lax.axis_index("c")
    s = lax.axis_index("s")
    r0 = s * _TILE_ROWS

    for off, sz in _ROW_CHUNKS:
      pltpu.sync_copy(z128.at[pl.ds(0, sz)], acc.at[pl.ds(r0 + off, sz)])

    @pl.when(s == 15)
    def _():
      pltpu.sync_copy(z128.at[pl.ds(0, _EXTRA_ALL)],
                      acc.at[pl.ds(_EXTRA_R0, _EXTRA_ALL)])

    plsc.subcore_barrier()

    def chunk_body(k, carry):
      base = s * NBLK + k * _ICH
      pltpu.sync_copy(src2.at[c, pl.ds(base, _ICH)], isrc)
      pltpu.sync_copy(dst2.at[pl.ds(base, _ICH)], idst)
      pltpu.async_copy(h2.at[isrc.at[0]], g0, sem0)
      pltpu.async_copy(h2.at[isrc.at[1]], g1, sem1)

      def body(i, cc):
        b = 2 * i
        pltpu.make_async_copy(h2.at[isrc.at[b]], g0, sem0).wait()
        pltpu.make_async_copy(h2.at[isrc.at[b + 1]], g1, sem1).wait()
        pltpu.async_copy(g0, acc.at[idst.at[b]], ssem0, add=True)
        pltpu.async_copy(g1, acc.at[idst.at[b + 1]], ssem1, add=True)
        pltpu.make_async_copy(g0, acc.at[idst.at[0]], ssem0).wait()
        pltpu.make_async_copy(g1, acc.at[idst.at[0]], ssem1).wait()

        @pl.when(i < _ICH // 2 - 1)
        def _():
          pltpu.async_copy(h2.at[isrc.at[b + 2]], g0, sem0)
          pltpu.async_copy(h2.at[isrc.at[b + 3]], g1, sem1)
        return cc

      lax.fori_loop(0, _ICH // 2, body, 0)
      return carry

    lax.fori_loop(0, NBLK // _ICH, chunk_body, 0)
    plsc.subcore_barrier()

    for off, sz in _ROW_CHUNKS:
      pltpu.sync_copy(acc.at[pl.ds(r0 + off, sz)],
                      agg_out.at[c, pl.ds(r0 + off, sz)])

    @pl.when(s == 15)
    def _():
      pltpu.sync_copy(acc.at[pl.ds(_EXTRA_R0, _EXTRA_OUT)],
                      agg_out.at[c, pl.ds(_EXTRA_R0, _EXTRA_OUT)])

  return agg


_CBLK = EP // (32 * BLK)


def _sc_count_build():
  mesh = plsc.VectorSubcoreMesh(core_axis_name="c", subcore_axis_name="s")
  out_type = [jax.ShapeDtypeStruct((2, N, 16), jnp.float32)]
  scratch = [
      pltpu.VMEM((_CBLK, BLK), jnp.int32),
      pltpu.VMEM((BLK, 16), jnp.float32),
      pltpu.VMEM_SHARED((N + PAD_ROWS, 16), jnp.float32),
  ]

  @functools.partial(pl.kernel, mesh=mesh, out_type=out_type,
                     scratch_types=scratch)
  def count(dst2, aux16, cnt_out, idst, ones_v, cacc):
    c = lax.axis_index("c")
    s = lax.axis_index("s")
    r0 = s * _TILE_ROWS

    for off, sz in _ROW_CHUNKS:
      pltpu.sync_copy(aux16.at[pl.ds(0, sz)], cacc.at[pl.ds(r0 + off, sz)])

    @pl.when(s == 15)
    def _():
      pltpu.sync_copy(aux16.at[pl.ds(0, _EXTRA_ALL)],
                      cacc.at[pl.ds(_EXTRA_R0, _EXTRA_ALL)])

    pltpu.sync_copy(aux16.at[pl.ds(128, BLK)], ones_v)
    w = c * 16 + s
    pltpu.sync_copy(dst2.at[pl.ds(w * _CBLK, _CBLK)], idst)
    plsc.subcore_barrier()

    def body(i, carry):
      pltpu.sync_copy(ones_v, cacc.at[idst.at[i]], add=True)
      return carry

    lax.fori_loop(0, _CBLK, body, 0)
    plsc.subcore_barrier()

    for off, sz in _ROW_CHUNKS:
      pltpu.sync_copy(cacc.at[pl.ds(r0 + off, sz)],
                      cnt_out.at[c, pl.ds(r0 + off, sz)])

    @pl.when(s == 15)
    def _():
      pltpu.sync_copy(cacc.at[pl.ds(_EXTRA_R0, _EXTRA_OUT)],
                      cnt_out.at[c, pl.ds(_EXTRA_R0, _EXTRA_OUT)])

  return count


@functools.cache
def _sc_agg_get():
  return _sc_agg_build()


@functools.cache
def _sc_count_get():
  return _sc_count_build()


def _sc_agg(*args):
  return _sc_agg_get()(*args)


def _sc_count(*args):
  return _sc_count_get()(*args)


def _gelu(x):
  return 0.5 * x * (1.0 + lax.erf(x * (2.0 ** -0.5)))


def _in_proj_body(x_ref, mask_ref, w_ref, b_ref, g_ref, beta_ref, out_ref):
  i = pl.program_id(0)
  row = i * BN + lax.broadcasted_iota(jnp.int32, (BN, 1), 0)
  xb = jnp.where(row == 0, mask_ref[...], x_ref[...])
  h = lax.dot_general(xb, w_ref[...], (((1,), (1,)), ((), ())),
                      preferred_element_type=jnp.float32) + b_ref[...]
  mu = jnp.mean(h, axis=1, keepdims=True)
  d = h - mu
  var = jnp.mean(d * d, axis=1, keepdims=True)
  hn = _gelu(d / jnp.sqrt(var + 1e-5) * g_ref[...] + beta_ref[...])
  out_ref[0] = hn[:, :128]
  out_ref[1] = hn[:, 128:]


def _in_proj(x, mask_token, w_in, b_in, ln_g, ln_b):
  return pl.pallas_call(
      _in_proj_body,
      grid=(GRID,),
      in_specs=[
          pl.BlockSpec((BN, IN_DIM), lambda i: (i, 0)),
          pl.BlockSpec((1, IN_DIM), lambda i: (0, 0)),
          pl.BlockSpec((HID, IN_DIM), lambda i: (0, 0)),
          pl.BlockSpec((1, HID), lambda i: (0, 0)),
          pl.BlockSpec((1, HID), lambda i: (0, 0)),
          pl.BlockSpec((1, HID), lambda i: (0, 0)),
      ],
      out_specs=pl.BlockSpec((2, BN, 128), lambda i: (0, i, 0)),
      out_shape=jax.ShapeDtypeStruct((2, N, 128), jnp.float32),
  )(x, mask_token.reshape(1, IN_DIM), w_in, b_in.reshape(1, HID),
    ln_g.reshape(1, HID), ln_b.reshape(1, HID))


def _layer_body(final, agg_ref, cnt_ref, h_ref, wl_ref, wr_ref, bl_ref,
                g_ref, b_ref, *rest):
  if final:
    wout_ref, bout_ref, out_ref, t_s, s1, s2 = rest
  else:
    out_ref, t_s, s1, s2 = rest
  i = pl.program_id(0)

  @pl.when(i < GRID)
  def _():
    @pl.when(i == 0)
    def _():
      s1[...] = jnp.zeros_like(s1)
      s2[...] = jnp.zeros_like(s2)

    cm = jnp.maximum(cnt_ref[0, :, :1] + cnt_ref[1, :, :1], 1.0)
    dn = (((1,), (1,)), ((), ()))
    mean = jnp.concatenate([agg_ref[0], agg_ref[1]], axis=1) / cm
    hf = jnp.concatenate([h_ref[0], h_ref[1]], axis=1)
    t = (lax.dot_general(mean, wl_ref[...], dn,
                         preferred_element_type=jnp.float32)
         + lax.dot_general(hf, wr_ref[...], dn,
                           preferred_element_type=jnp.float32)
         + bl_ref[...])
    t_s[pl.ds(i * BN, BN)] = t
    s1[...] += jnp.sum(t, axis=0, keepdims=True)
    s2[...] += jnp.sum(t * t, axis=0, keepdims=True)

  @pl.when(i >= GRID)
  def _():
    j = i - GRID
    t = t_s[pl.ds(j * BN, BN)]
    mu = s1[...] / N
    var = s2[...] / N - mu * mu
    tn = (t - mu) / jnp.sqrt(var + 1e-5) * g_ref[...] + b_ref[...]
    hn = jnp.concatenate([h_ref[0], h_ref[1]], axis=1) + _gelu(tn)
    if final:
      out_ref[...] = lax.dot_general(
          hn, wout_ref[...], (((1,), (1,)), ((), ())),
          preferred_element_type=jnp.float32) + bout_ref[...]
    else:
      out_ref[0] = hn[:, :128]
      out_ref[1] = hn[:, 128:]


def _layer(agg, cnt, h, wl, wr, bl, bn_g, bn_b, w_out=None, b_out=None):
  final = w_out is not None
  in_specs = [
      pl.BlockSpec((2, BN, 128), lambda i: (0, jnp.minimum(i, GRID - 1), 0)),
      pl.BlockSpec((2, BN, 16), lambda i: (0, jnp.minimum(i, GRID - 1), 0)),
      pl.BlockSpec((2, BN, 128),
                   lambda i: (0, jnp.where(i < GRID, i, i - GRID), 0)),
      pl.BlockSpec((HID, HID), lambda i: (0, 0)),
      pl.BlockSpec((HID, HID), lambda i: (0, 0)),
      pl.BlockSpec((1, HID), lambda i: (0, 0)),
      pl.BlockSpec((1, HID), lambda i: (0, 0)),
      pl.BlockSpec((1, HID), lambda i: (0, 0)),
  ]
  args = [agg, cnt, h, wl, wr, bl, bn_g.reshape(1, HID), bn_b.reshape(1, HID)]
  if final:
    in_specs += [
        pl.BlockSpec((OUT_DIM, HID), lambda i: (0, 0)),
        pl.BlockSpec((1, OUT_DIM), lambda i: (0, 0)),
    ]
    args += [w_out, b_out.reshape(1, OUT_DIM)]
    out_specs = pl.BlockSpec((BN, OUT_DIM),
                             lambda i: (jnp.maximum(i - GRID, 0), 0))
    out_shape = jax.ShapeDtypeStruct((N, OUT_DIM), jnp.float32)
  else:
    out_specs = pl.BlockSpec(
        (2, BN, 128), lambda i: (0, jnp.maximum(i - GRID, 0), 0))
    out_shape = jax.ShapeDtypeStruct((2, N, 128), jnp.float32)
  return pl.pallas_call(
      functools.partial(_layer_body, final),
      grid=(2 * GRID,),
      in_specs=in_specs,
      out_specs=out_specs,
      out_shape=out_shape,
      scratch_shapes=[
          pltpu.VMEM((N, HID), jnp.float32),
          pltpu.VMEM((1, HID), jnp.float32),
          pltpu.VMEM((1, HID), jnp.float32),
      ],
  )(*args)


def kernel(x, edge_index, mask_token, W_in, b_in, ln_g, ln_b, Wl, bl, Wr,
           bn_g, bn_b, W_out, b_out):
  src = edge_index[0]
  dst = edge_index[1]
  padi = jnp.arange(EP - E, dtype=jnp.int32)
  src_p = jnp.concatenate([src, padi % N])
  dst_p = jnp.concatenate([dst, N + (padi % PAD_ROWS)])
  src2 = jnp.stack([src_p, src_p + N]).reshape(2, EP // BLK, BLK)
  dst2 = dst_p.reshape(EP // BLK, BLK)
  z128 = jnp.zeros((128, 128), jnp.float32)
  aux16 = jnp.concatenate(
      [jnp.zeros((128, 16), jnp.float32), jnp.ones((128, 16), jnp.float32)])

  h = _in_proj(x, mask_token, W_in, b_in, ln_g, ln_b)
  cnt, = _sc_count(dst2, aux16)
  out = None
  for i in range(L):
    agg, = _sc_agg(h.reshape(2 * N, 128), src2, dst2, z128)
    if i == L - 1:
      out = _layer(agg, cnt, h, Wl[i], Wr[i], bl[i].reshape(1, HID),
                   bn_g[i], bn_b[i], W_out, b_out)
    else:
      h = _layer(agg, cnt, h, Wl[i], Wr[i], bl[i].reshape(1, HID),
                 bn_g[i], bn_b[i])
  return out


# speedup vs baseline: 1.3037x; 1.3037x over previous
import functools

import jax
import jax.numpy as jnp
from jax import lax
from jax.experimental import pallas as pl
from jax.experimental.pallas import tpu as pltpu
from jax.experimental.pallas import tpu_sc as plsc

N = 10000
E = 320000
IN_DIM = 128
HID = 256
OUT_DIM = 128
L = 3

EP = 327680
BLK = 128
NBLK = EP // (16 * BLK)
BN = 2000
GRID = N // BN
PAD_ROWS = 8

_TILE_ROWS = 624
_ROW_CHUNKS = ((0, 128), (128, 128), (256, 128), (384, 128), (512, 112))
_EXTRA_R0 = 16 * _TILE_ROWS
_EXTRA_ALL = 24
_EXTRA_OUT = 16
_ICH = 32


def _sc_agg_build():
  mesh = plsc.VectorSubcoreMesh(core_axis_name="c", subcore_axis_name="s")
  out_type = [jax.ShapeDtypeStruct((2, N, 128), jnp.float32)]
  scratch = [
      pltpu.VMEM((_ICH, BLK), jnp.int32),
      pltpu.VMEM((_ICH, BLK), jnp.int32),
      pltpu.VMEM((BLK, 128), jnp.float32),
      pltpu.VMEM((BLK, 128), jnp.float32),
      pltpu.VMEM_SHARED((N + PAD_ROWS, 128), jnp.float32),
      pltpu.SemaphoreType.DMA,
      pltpu.SemaphoreType.DMA,
  ]

  @functools.partial(pl.kernel, mesh=mesh, out_type=out_type,
                     scratch_types=scratch)
  def agg(h2, src2, dst2, z128, *refs):
    agg_out, isrc, idst, g0, g1, acc, sem0, sem1 = refs
    c = lax.axis_index("c")
    s = lax.axis_index("s")
    r0 = s * _TILE_ROWS

    for off, sz in _ROW_CHUNKS:
      pltpu.sync_copy(z128.at[pl.ds(0, sz)], acc.at[pl.ds(r0 + off, sz)])

    @pl.when(s == 15)
    def _():
      pltpu.sync_copy(z128.at[pl.ds(0, _EXTRA_ALL)],
                      acc.at[pl.ds(_EXTRA_R0, _EXTRA_ALL)])

    plsc.subcore_barrier()

    def chunk_body(k, carry):
      base = s * NBLK + k * _ICH
      pltpu.sync_copy(src2.at[c, pl.ds(base, _ICH)], isrc)
      pltpu.sync_copy(dst2.at[pl.ds(base, _ICH)], idst)
      pltpu.async_copy(h2.at[isrc.at[0]], g0, sem0)

      def body(i, cc):
        b = 2 * i
        pltpu.async_copy(h2.at[isrc.at[b + 1]], g1, sem1)
        pltpu.make_async_copy(h2.at[isrc.at[b]], g0, sem0).wait()
        pltpu.sync_copy(g0, acc.at[idst.at[b]], add=True)

        @pl.when(i < _ICH // 2 - 1)
        def _():
          pltpu.async_copy(h2.at[isrc.at[b + 2]], g0, sem0)

        pltpu.make_async_copy(h2.at[isrc.at[b + 1]], g1, sem1).wait()
        pltpu.sync_copy(g1, acc.at[idst.at[b + 1]], add=True)
        return cc

      lax.fori_loop(0, _ICH // 2, body, 0)
      return carry

    lax.fori_loop(0, NBLK // _ICH, chunk_body, 0)
    plsc.subcore_barrier()

    for off, sz in _ROW_CHUNKS:
      pltpu.sync_copy(acc.at[pl.ds(r0 + off, sz)],
                      agg_out.at[c, pl.ds(r0 + off, sz)])

    @pl.when(s == 15)
    def _():
      pltpu.sync_copy(acc.at[pl.ds(_EXTRA_R0, _EXTRA_OUT)],
                      agg_out.at[c, pl.ds(_EXTRA_R0, _EXTRA_OUT)])

  return agg


_CBLK = EP // (32 * BLK)


def _sc_count_build():
  mesh = plsc.VectorSubcoreMesh(core_axis_name="c", subcore_axis_name="s")
  out_type = [jax.ShapeDtypeStruct((2, N, 16), jnp.float32)]
  scratch = [
      pltpu.VMEM((_CBLK, BLK), jnp.int32),
      pltpu.VMEM((BLK, 16), jnp.float32),
      pltpu.VMEM_SHARED((N + PAD_ROWS, 16), jnp.float32),
  ]

  @functools.partial(pl.kernel, mesh=mesh, out_type=out_type,
                     scratch_types=scratch)
  def count(dst2, aux16, cnt_out, idst, ones_v, cacc):
    c = lax.axis_index("c")
    s = lax.axis_index("s")
    r0 = s * _TILE_ROWS

    for off, sz in _ROW_CHUNKS:
      pltpu.sync_copy(aux16.at[pl.ds(0, sz)], cacc.at[pl.ds(r0 + off, sz)])

    @pl.when(s == 15)
    def _():
      pltpu.sync_copy(aux16.at[pl.ds(0, _EXTRA_ALL)],
                      cacc.at[pl.ds(_EXTRA_R0, _EXTRA_ALL)])

    pltpu.sync_copy(aux16.at[pl.ds(128, BLK)], ones_v)
    w = c * 16 + s
    pltpu.sync_copy(dst2.at[pl.ds(w * _CBLK, _CBLK)], idst)
    plsc.subcore_barrier()

    def body(i, carry):
      pltpu.sync_copy(ones_v, cacc.at[idst.at[i]], add=True)
      return carry

    lax.fori_loop(0, _CBLK, body, 0)
    plsc.subcore_barrier()

    for off, sz in _ROW_CHUNKS:
      pltpu.sync_copy(cacc.at[pl.ds(r0 + off, sz)],
                      cnt_out.at[c, pl.ds(r0 + off, sz)])

    @pl.when(s == 15)
    def _():
      pltpu.sync_copy(cacc.at[pl.ds(_EXTRA_R0, _EXTRA_OUT)],
                      cnt_out.at[c, pl.ds(_EXTRA_R0, _EXTRA_OUT)])

  return count


@functools.cache
def _sc_agg_get():
  return _sc_agg_build()


@functools.cache
def _sc_count_get():
  return _sc_count_build()


def _sc_agg(*args):
  return _sc_agg_get()(*args)


def _sc_count(*args):
  return _sc_count_get()(*args)


def _gelu(x):
  return 0.5 * x * (1.0 + lax.erf(x * (2.0 ** -0.5)))


def _in_proj_body(x_ref, mask_ref, w_ref, b_ref, g_ref, beta_ref, out_ref):
  i = pl.program_id(0)
  row = i * BN + lax.broadcasted_iota(jnp.int32, (BN, 1), 0)
  xb = jnp.where(row == 0, mask_ref[...], x_ref[...])
  h = lax.dot_general(xb, w_ref[...], (((1,), (1,)), ((), ())),
                      preferred_element_type=jnp.float32) + b_ref[...]
  mu = jnp.mean(h, axis=1, keepdims=True)
  d = h - mu
  var = jnp.mean(d * d, axis=1, keepdims=True)
  hn = _gelu(d / jnp.sqrt(var + 1e-5) * g_ref[...] + beta_ref[...])
  out_ref[0] = hn[:, :128]
  out_ref[1] = hn[:, 128:]


def _in_proj(x, mask_token, w_in, b_in, ln_g, ln_b):
  return pl.pallas_call(
      _in_proj_body,
      grid=(GRID,),
      in_specs=[
          pl.BlockSpec((BN, IN_DIM), lambda i: (i, 0)),
          pl.BlockSpec((1, IN_DIM), lambda i: (0, 0)),
          pl.BlockSpec((HID, IN_DIM), lambda i: (0, 0)),
          pl.BlockSpec((1, HID), lambda i: (0, 0)),
          pl.BlockSpec((1, HID), lambda i: (0, 0)),
          pl.BlockSpec((1, HID), lambda i: (0, 0)),
      ],
      out_specs=pl.BlockSpec((2, BN, 128), lambda i: (0, i, 0)),
      out_shape=jax.ShapeDtypeStruct((2, N, 128), jnp.float32),
  )(x, mask_token.reshape(1, IN_DIM), w_in, b_in.reshape(1, HID),
    ln_g.reshape(1, HID), ln_b.reshape(1, HID))


def _layer_body(final, agg_ref, cnt_ref, h_ref, wl_ref, wr_ref, bl_ref,
                g_ref, b_ref, *rest):
  if final:
    wout_ref, bout_ref, out_ref, t_s, s1, s2 = rest
  else:
    out_ref, t_s, s1, s2 = rest
  i = pl.program_id(0)

  @pl.when(i < GRID)
  def _():
    @pl.when(i == 0)
    def _():
      s1[...] = jnp.zeros_like(s1)
      s2[...] = jnp.zeros_like(s2)

    cm = jnp.maximum(cnt_ref[0, :, :1] + cnt_ref[1, :, :1], 1.0)
    dn = (((1,), (1,)), ((), ()))
    mean = jnp.concatenate([agg_ref[0], agg_ref[1]], axis=1) / cm
    hf = jnp.concatenate([h_ref[0], h_ref[1]], axis=1)
    t = (lax.dot_general(mean, wl_ref[...], dn,
                         preferred_element_type=jnp.float32)
         + lax.dot_general(hf, wr_ref[...], dn,
                           preferred_element_type=jnp.float32)
         + bl_ref[...])
    t_s[pl.ds(i * BN, BN)] = t
    s1[...] += jnp.sum(t, axis=0, keepdims=True)
    s2[...] += jnp.sum(t * t, axis=0, keepdims=True)

  @pl.when(i >= GRID)
  def _():
    j = i - GRID
    t = t_s[pl.ds(j * BN, BN)]
    mu = s1[...] / N
    var = s2[...] / N - mu * mu
    tn = (t - mu) / jnp.sqrt(var + 1e-5) * g_ref[...] + b_ref[...]
    hn = jnp.concatenate([h_ref[0], h_ref[1]], axis=1) + _gelu(tn)
    if final:
      out_ref[...] = lax.dot_general(
          hn, wout_ref[...], (((1,), (1,)), ((), ())),
          preferred_element_type=jnp.float32) + bout_ref[...]
    else:
      out_ref[0] = hn[:, :128]
      out_ref[1] = hn[:, 128:]


def _layer(agg, cnt, h, wl, wr, bl, bn_g, bn_b, w_out=None, b_out=None):
  final = w_out is not None
  in_specs = [
      pl.BlockSpec((2, BN, 128), lambda i: (0, jnp.minimum(i, GRID - 1), 0)),
      pl.BlockSpec((2, BN, 16), lambda i: (0, jnp.minimum(i, GRID - 1), 0)),
      pl.BlockSpec((2, BN, 128),
                   lambda i: (0, jnp.where(i < GRID, i, i - GRID), 0)),
      pl.BlockSpec((HID, HID), lambda i: (0, 0)),
      pl.BlockSpec((HID, HID), lambda i: (0, 0)),
      pl.BlockSpec((1, HID), lambda i: (0, 0)),
      pl.BlockSpec((1, HID), lambda i: (0, 0)),
      pl.BlockSpec((1, HID), lambda i: (0, 0)),
  ]
  args = [agg, cnt, h, wl, wr, bl, bn_g.reshape(1, HID), bn_b.reshape(1, HID)]
  if final:
    in_specs += [
        pl.BlockSpec((OUT_DIM, HID), lambda i: (0, 0)),
        pl.BlockSpec((1, OUT_DIM), lambda i: (0, 0)),
    ]
    args += [w_out, b_out.reshape(1, OUT_DIM)]
    out_specs = pl.BlockSpec((BN, OUT_DIM),
                             lambda i: (jnp.maximum(i - GRID, 0), 0))
    out_shape = jax.ShapeDtypeStruct((N, OUT_DIM), jnp.float32)
  else:
    out_specs = pl.BlockSpec(
        (2, BN, 128), lambda i: (0, jnp.maximum(i - GRID, 0), 0))
    out_shape = jax.ShapeDtypeStruct((2, N, 128), jnp.float32)
  return pl.pallas_call(
      functools.partial(_layer_body, final),
      grid=(2 * GRID,),
      in_specs=in_specs,
      out_specs=out_specs,
      out_shape=out_shape,
      scratch_shapes=[
          pltpu.VMEM((N, HID), jnp.float32),
          pltpu.VMEM((1, HID), jnp.float32),
          pltpu.VMEM((1, HID), jnp.float32),
      ],
  )(*args)


def kernel(x, edge_index, mask_token, W_in, b_in, ln_g, ln_b, Wl, bl, Wr,
           bn_g, bn_b, W_out, b_out):
  src = edge_index[0]
  dst = edge_index[1]
  padi = jnp.arange(EP - E, dtype=jnp.int32)
  src_p = jnp.concatenate([src, padi % N])
  dst_p = jnp.concatenate([dst, N + (padi % PAD_ROWS)])
  src2 = jnp.stack([src_p, src_p + N]).reshape(2, EP // BLK, BLK)
  dst2 = dst_p.reshape(EP // BLK, BLK)
  z128 = jnp.zeros((128, 128), jnp.float32)
  aux16 = jnp.concatenate(
      [jnp.zeros((128, 16), jnp.float32), jnp.ones((128, 16), jnp.float32)])

  h = _in_proj(x, mask_token, W_in, b_in, ln_g, ln_b)
  cnt, = _sc_count(dst2, aux16)
  out = None
  for i in range(L):
    agg, = _sc_agg(h.reshape(2 * N, 128), src2, dst2, z128)
    if i == L - 1:
      out = _layer(agg, cnt, h, Wl[i], Wr[i], bl[i].reshape(1, HID),
                   bn_g[i], bn_b[i], W_out, b_out)
    else:
      h = _layer(agg, cnt, h, Wl[i], Wr[i], bl[i].reshape(1, HID),
                 bn_g[i], bn_b[i])
  return out
